# baseline (speedup 1.0000x reference)
import functools

import jax
import jax.numpy as jnp
from jax import lax
from jax.experimental import pallas as pl
from jax.experimental.pallas import tpu as pltpu
from jax.experimental.pallas import tpu_sc as plsc

NUM_ATOMS = 6_400_000
NUM_MOL = 100_000

NC, NS = 2, 16
NW = NC * NS
APT = NUM_ATOMS // NW
CHUNK = 8000
FULL_CHUNKS = APT // CHUNK
NSLOTS = 5
ACC_PAD = 100_352
SLICE = ACC_PAD // NS
PIECE = 512
WIN = 16_384


def _sc_body(vals_hbm, idx_hbm, out_hbm, vbufs, ibufs, zbuf, ebuf, pbuf,
             idbuf, wacc, acc, sems, sem_s):
    c = lax.axis_index("c")
    s = lax.axis_index("s")
    w = c * NS + s
    a0 = w * APT

    def start_load(i, b):
        off = a0 + i * CHUNK
        pltpu.async_copy(vals_hbm.at[pl.ds(off, CHUNK)],
                         vbufs[b].at[pl.ds(0, CHUNK)], sems[b])
        pltpu.async_copy(idx_hbm.at[pl.ds(off, CHUNK)],
                         ibufs[b].at[pl.ds(0, CHUNK)], sems[b])

    for b in range(NSLOTS - 1):
        start_load(b, b)

    lane = lax.iota(jnp.int32, 16)
    lt15 = lane < 15
    is15 = lane == 15
    zeros16 = jnp.zeros((16,), jnp.float32)

    def _zero(i, carry):
        zbuf[pl.ds(i * 16, 16)] = zeros16
        return carry
    lax.fori_loop(0, SLICE // 16, _zero, None)
    pltpu.sync_copy(zbuf, acc.at[pl.ds(s * SLICE, SLICE)])

    for b in range(NSLOTS):
        vbufs[b][pl.ds(CHUNK, 16)] = zeros16
        ibufs[b][pl.ds(CHUNK, 16)] = jnp.zeros((16,), jnp.int32)

    pltpu.sync_copy(idx_hbm.at[pl.ds(a0, 8)], ebuf.at[pl.ds(0, 8)])
    pltpu.sync_copy(idx_hbm.at[pl.ds(a0 + APT - 8, 8)], ebuf.at[pl.ds(8, 8)])
    ev = ebuf[pl.ds(0, 16)]
    mlo = ev[0]
    mhi = ev[15]
    span = mhi - mlo
    fits = span <= (WIN - PIECE)
    npieces = span // PIECE + 1

    @pl.when(fits)
    def _():
        def _wzero(k, carry):
            for j in range(PIECE // 16):
                wacc[pl.ds(k * PIECE + 16 * j, 16)] = zeros16
            return carry
        lax.fori_loop(0, npieces, _wzero, None)

    plsc.subcore_barrier()

    def wait_load(i, b):
        off = a0 + i * CHUNK
        pltpu.make_async_copy(vals_hbm.at[pl.ds(off, CHUNK)],
                              vbufs[b].at[pl.ds(0, CHUNK)], sems[b]).wait()
        pltpu.make_async_copy(idx_hbm.at[pl.ds(off, CHUNK)],
                              ibufs[b].at[pl.ds(0, CHUNK)], sems[b]).wait()

    def process(b):
        vb, ib = vbufs[b], ibufs[b]

        @pl.when(fits)
        def _fast():
            @plsc.parallel_loop(0, CHUNK // 16, 1, unroll=10)
            def _inner(m):
                v = vb[pl.ds(16 * m, 16)]
                ic = ib[pl.ds(16 * m, 16)]
                inx = ib[pl.ds(16 * m + 1, 16)]
                p = plsc.cumsum(v)
                trans = (ic != inx) & lt15
                endm = trans | is15
                plsc.addupdate_scatter(wacc, [ic - mlo], p, mask=endm)
                plsc.addupdate_scatter(wacc, [inx - mlo], -p, mask=trans)

        @pl.when(jnp.logical_not(fits))
        def _slow():
            pltpu.async_copy(vb, acc.at[ib], sem_s, add=True).wait()

    def _ring(j, carry):
        for b in range(NSLOTS):
            i = NSLOTS * j + b

            @pl.when(i + NSLOTS - 1 < FULL_CHUNKS)
            def _():
                start_load(i + NSLOTS - 1, (b + NSLOTS - 1) % NSLOTS)
            wait_load(i, b)
            process(b)
        return carry
    lax.fori_loop(0, FULL_CHUNKS // NSLOTS, _ring, None)

    @pl.when(fits)
    def _():
        def _merge(k, carry):
            base = k * PIECE
            for j in range(PIECE // 16):
                pbuf[pl.ds(16 * j, 16)] = wacc[pl.ds(base + 16 * j, 16)]
                idbuf[pl.ds(16 * j, 16)] = mlo + base + 16 * j + lane
            pltpu.async_copy(pbuf, acc.at[idbuf], sem_s, add=True).wait()
            return carry
        lax.fori_loop(0, npieces, _merge, None)

    plsc.subcore_barrier()
    pltpu.sync_copy(acc.at[pl.ds(s * SLICE, SLICE)],
                    out_hbm.at[c, pl.ds(s * SLICE, SLICE)])


def _wrapped(vals_hbm, idx_hbm, out_hbm, v0, v1, v2, v3, v4,
             i0, i1, i2, i3, i4, zbuf, ebuf, pbuf, idbuf, wacc, acc,
             s0, s1, s2, s3, s4, sem_s):
    _sc_body(vals_hbm, idx_hbm, out_hbm, (v0, v1, v2, v3, v4),
             (i0, i1, i2, i3, i4), zbuf, ebuf, pbuf, idbuf, wacc, acc,
             (s0, s1, s2, s3, s4), sem_s)


_sc_call = functools.partial(
    pl.kernel,
    out_type=jax.ShapeDtypeStruct((NC, ACC_PAD), jnp.float32),
    mesh=plsc.VectorSubcoreMesh(core_axis_name="c", subcore_axis_name="s"),
    compiler_params=pltpu.CompilerParams(needs_layout_passes=False),
    scratch_types=(
        [pltpu.VMEM((CHUNK + 16,), jnp.float32) for _ in range(5)]
        + [pltpu.VMEM((CHUNK + 16,), jnp.int32) for _ in range(5)]
        + [
            pltpu.VMEM((SLICE,), jnp.float32),
            pltpu.VMEM((16,), jnp.int32),
            pltpu.VMEM((PIECE,), jnp.float32),
            pltpu.VMEM((PIECE,), jnp.int32),
            pltpu.VMEM((WIN,), jnp.float32),
            pltpu.VMEM_SHARED((ACC_PAD,), jnp.float32),
        ]
        + [pltpu.SemaphoreType.DMA for _ in range(6)]
    ),
)(_wrapped)


def kernel(atom_specific_values, index):
    partials = _sc_call(atom_specific_values, index.astype(jnp.int32))
    return (partials[0] + partials[1])[:NUM_MOL]


# speedup vs baseline: 1.0050x; 1.0050x over previous
import functools

import jax
import jax.numpy as jnp
from jax import lax
from jax.experimental import pallas as pl
from jax.experimental.pallas import tpu as pltpu
from jax.experimental.pallas import tpu_sc as plsc

NUM_ATOMS = 6_400_000
NUM_MOL = 100_000

NC, NS = 2, 16
NW = NC * NS
APT = NUM_ATOMS // NW
CHUNK = 8000
FULL_CHUNKS = APT // CHUNK
NSLOTS = 5
ACC_PAD = 100_352
SLICE = ACC_PAD // NS
PIECE = 512
WIN = 16_384


def _sc_body(vals_hbm, idx_hbm, out_hbm, vbufs, ibufs, zbuf, ebuf, pbuf,
             idbuf, wacc, acc, sems, sem_s):
    c = lax.axis_index("c")
    s = lax.axis_index("s")
    w = c * NS + s
    a0 = w * APT

    def start_load(i, b):
        off = a0 + i * CHUNK
        pltpu.async_copy(vals_hbm.at[pl.ds(off, CHUNK)],
                         vbufs[b].at[pl.ds(0, CHUNK)], sems[b])
        pltpu.async_copy(idx_hbm.at[pl.ds(off, CHUNK)],
                         ibufs[b].at[pl.ds(0, CHUNK)], sems[b])

    for b in range(NSLOTS - 1):
        start_load(b, b)

    lane = lax.iota(jnp.int32, 16)
    lt15 = lane < 15
    is15 = lane == 15
    zeros16 = jnp.zeros((16,), jnp.float32)

    def _zero(i, carry):
        zbuf[pl.ds(i * 16, 16)] = zeros16
        return carry
    lax.fori_loop(0, SLICE // 16, _zero, None)
    pltpu.sync_copy(zbuf, acc.at[pl.ds(s * SLICE, SLICE)])

    for b in range(NSLOTS):
        vbufs[b][pl.ds(CHUNK, 16)] = zeros16
        ibufs[b][pl.ds(CHUNK, 16)] = jnp.zeros((16,), jnp.int32)

    pltpu.sync_copy(idx_hbm.at[pl.ds(a0, 8)], ebuf.at[pl.ds(0, 8)])
    pltpu.sync_copy(idx_hbm.at[pl.ds(a0 + APT - 8, 8)], ebuf.at[pl.ds(8, 8)])
    ev = ebuf[pl.ds(0, 16)]
    mlo = ev[0]
    mhi = ev[15]
    span = mhi - mlo
    fits = span <= (WIN - PIECE)
    npieces = span // PIECE + 1

    @pl.when(fits)
    def _():
        def _wzero(k, carry):
            for j in range(PIECE // 16):
                wacc[pl.ds(k * PIECE + 16 * j, 16)] = zeros16
            return carry
        lax.fori_loop(0, npieces, _wzero, None)

    plsc.subcore_barrier()

    def wait_load(i, b):
        off = a0 + i * CHUNK
        pltpu.make_async_copy(vals_hbm.at[pl.ds(off, CHUNK)],
                              vbufs[b].at[pl.ds(0, CHUNK)], sems[b]).wait()
        pltpu.make_async_copy(idx_hbm.at[pl.ds(off, CHUNK)],
                              ibufs[b].at[pl.ds(0, CHUNK)], sems[b]).wait()

    def process(b):
        vb, ib = vbufs[b], ibufs[b]

        @pl.when(fits)
        def _fast():
            @plsc.parallel_loop(0, CHUNK // 16, 1, unroll=5)
            def _inner(m):
                v = vb[pl.ds(16 * m, 16)]
                ic = ib[pl.ds(16 * m, 16)]
                inx = ib[pl.ds(16 * m + 1, 16)]
                p = plsc.cumsum(v)
                trans = (ic != inx) & lt15
                endm = trans | is15
                plsc.addupdate_scatter(wacc, [ic - mlo], p, mask=endm)
                plsc.addupdate_scatter(wacc, [inx - mlo], -p, mask=trans)

        @pl.when(jnp.logical_not(fits))
        def _slow():
            pltpu.async_copy(vb, acc.at[ib], sem_s, add=True).wait()

    def _ring(j, carry):
        for b in range(NSLOTS):
            i = NSLOTS * j + b

            @pl.when(i + NSLOTS - 1 < FULL_CHUNKS)
            def _():
                start_load(i + NSLOTS - 1, (b + NSLOTS - 1) % NSLOTS)
            wait_load(i, b)
            process(b)
        return carry
    lax.fori_loop(0, FULL_CHUNKS // NSLOTS, _ring, None)

    @pl.when(fits)
    def _():
        def _merge(k, carry):
            base = k * PIECE
            for j in range(PIECE // 16):
                pbuf[pl.ds(16 * j, 16)] = wacc[pl.ds(base + 16 * j, 16)]
                idbuf[pl.ds(16 * j, 16)] = mlo + base + 16 * j + lane
            pltpu.async_copy(pbuf, acc.at[idbuf], sem_s, add=True).wait()
            return carry
        lax.fori_loop(0, npieces, _merge, None)

    plsc.subcore_barrier()
    pltpu.sync_copy(acc.at[pl.ds(s * SLICE, SLICE)],
                    out_hbm.at[c, pl.ds(s * SLICE, SLICE)])


def _wrapped(vals_hbm, idx_hbm, out_hbm, v0, v1, v2, v3, v4,
             i0, i1, i2, i3, i4, zbuf, ebuf, pbuf, idbuf, wacc, acc,
             s0, s1, s2, s3, s4, sem_s):
    _sc_body(vals_hbm, idx_hbm, out_hbm, (v0, v1, v2, v3, v4),
             (i0, i1, i2, i3, i4), zbuf, ebuf, pbuf, idbuf, wacc, acc,
             (s0, s1, s2, s3, s4), sem_s)


_sc_call = functools.partial(
    pl.kernel,
    out_type=jax.ShapeDtypeStruct((NC, ACC_PAD), jnp.float32),
    mesh=plsc.VectorSubcoreMesh(core_axis_name="c", subcore_axis_name="s"),
    compiler_params=pltpu.CompilerParams(needs_layout_passes=False),
    scratch_types=(
        [pltpu.VMEM((CHUNK + 16,), jnp.float32) for _ in range(5)]
        + [pltpu.VMEM((CHUNK + 16,), jnp.int32) for _ in range(5)]
        + [
            pltpu.VMEM((SLICE,), jnp.float32),
            pltpu.VMEM((16,), jnp.int32),
            pltpu.VMEM((PIECE,), jnp.float32),
            pltpu.VMEM((PIECE,), jnp.int32),
            pltpu.VMEM((WIN,), jnp.float32),
            pltpu.VMEM_SHARED((ACC_PAD,), jnp.float32),
        ]
        + [pltpu.SemaphoreType.DMA for _ in range(6)]
    ),
)(_wrapped)


def kernel(atom_specific_values, index):
    partials = _sc_call(atom_specific_values, index.astype(jnp.int32))
    return (partials[0] + partials[1])[:NUM_MOL]
